# baseline (speedup 1.0000x reference)
import functools

import jax
import jax.numpy as jnp
from jax import lax
from jax.experimental import pallas as pl
from jax.experimental.pallas import tpu as pltpu
from jax.experimental.pallas import tpu_sc as plsc

NC = 2
NS = 16
NW = NC * NS


def _make_sc_gather(V, D, B_src, B_dst):
    bs, bd = B_src // NW, B_dst // NW
    ch_s = min(bs, 128)
    ncs = bs // ch_s
    ch_d = min(bd, 128)
    ncd = bd // ch_d
    mesh = plsc.VectorSubcoreMesh(core_axis_name="c", subcore_axis_name="s")

    @functools.partial(
        pl.kernel,
        out_type=(
            jax.ShapeDtypeStruct((B_src, D), jnp.float32),
            jax.ShapeDtypeStruct((B_dst, D), jnp.float32),
        ),
        mesh=mesh,
        scratch_types=[
            pltpu.VMEM((ncs, ch_s), jnp.int32),
            pltpu.VMEM((bs, D), jnp.float32),
            pltpu.VMEM((ncd, ch_d), jnp.int32),
            pltpu.VMEM((bd, D), jnp.float32),
            pltpu.SemaphoreType.DMA,
            pltpu.SemaphoreType.DMA,
        ],
    )
    def gather(table, sidx, didx, src_out, dst_out,
               sidx_v, srows_v, didx_v, drows_v, gsem, wsem):
        wid = lax.axis_index("s") * NC + lax.axis_index("c")
        sbase = wid * bs
        dbase = wid * bd
        for j in range(ncs):
            pltpu.sync_copy(sidx.at[pl.ds(sbase + j * ch_s, ch_s)], sidx_v.at[j])
        for j in range(ncd):
            pltpu.sync_copy(didx.at[pl.ds(dbase + j * ch_d, ch_d)], didx_v.at[j])
        gathers = [
            pltpu.async_copy(table.at[sidx_v.at[j]],
                             srows_v.at[pl.ds(j * ch_s, ch_s)], gsem)
            for j in range(ncs)
        ] + [
            pltpu.async_copy(table.at[didx_v.at[j]],
                             drows_v.at[pl.ds(j * ch_d, ch_d)], gsem)
            for j in range(ncd)
        ]
        writebacks = []
        for j in range(ncs):
            gathers[j].wait()
            writebacks.append(pltpu.async_copy(
                srows_v.at[pl.ds(j * ch_s, ch_s)],
                src_out.at[pl.ds(sbase + j * ch_s, ch_s)], wsem))
        for j in range(ncd):
            gathers[ncs + j].wait()
            writebacks.append(pltpu.async_copy(
                drows_v.at[pl.ds(j * ch_d, ch_d)],
                dst_out.at[pl.ds(dbase + j * ch_d, ch_d)], wsem))
        for c in writebacks:
            c.wait()

    return gather


def _mm_big(dif, g, d2, wa, wb, bm=256):
    M, K = dif.shape
    D = g.shape[1]
    nm = M // bm

    def body(dif_ref, g_ref, d2_ref, wa_ref, wb_ref, out_ref):
        agg = jnp.dot(dif_ref[...], g_ref[...],
                      preferred_element_type=jnp.float32)
        out_ref[...] = jnp.maximum(
            jnp.dot(agg, wa_ref[...], preferred_element_type=jnp.float32)
            + jnp.dot(d2_ref[...], wb_ref[...],
                      preferred_element_type=jnp.float32),
            0.0)

    return pl.pallas_call(
        body,
        grid=(nm,),
        in_specs=[
            pl.BlockSpec((bm, K), lambda m: (m, 0)),
            pl.BlockSpec((K, D), lambda m: (0, 0)),
            pl.BlockSpec((bm, D), lambda m: (m, 0)),
            pl.BlockSpec((D, D), lambda m: (0, 0)),
            pl.BlockSpec((D, D), lambda m: (0, 0)),
        ],
        out_specs=pl.BlockSpec((bm, D), lambda m: (m, 0)),
        out_shape=jax.ShapeDtypeStruct((M, D), jnp.float32),
    )(dif, g, d2, wa, wb)


def _mm_small(dif, g, d1, wa, wb):
    M = dif.shape[0]
    D = g.shape[1]

    def body(dif_ref, g_ref, d_ref, wa_ref, wb_ref, out_ref):
        agg = jnp.dot(dif_ref[...], g_ref[...],
                      preferred_element_type=jnp.float32)
        out_ref[...] = jnp.maximum(
            jnp.dot(agg, wa_ref[...], preferred_element_type=jnp.float32)
            + jnp.dot(d_ref[...], wb_ref[...],
                      preferred_element_type=jnp.float32),
            0.0)

    return pl.pallas_call(
        body,
        out_shape=jax.ShapeDtypeStruct((M, D), jnp.float32),
    )(dif, g, d1, wa, wb)


def kernel(src_nodes, dstsrc2src_1, dstsrc2src_2, dstsrc2dst_1, dstsrc2dst_2,
           dif_mat_1, dif_mat_2, w1, w2):
    D = src_nodes.shape[1]
    w1a, w1b = w1[:D], w1[D:]
    w2a, w2b = w2[:D], w2[D:]

    gather1 = _make_sc_gather(src_nodes.shape[0], D,
                              dstsrc2src_2.shape[0], dstsrc2dst_2.shape[0])
    g2, d2 = gather1(src_nodes, dstsrc2src_2, dstsrc2dst_2)
    x = _mm_big(dif_mat_2, g2, d2, w1a, w1b)

    gather2 = _make_sc_gather(x.shape[0], D,
                              dstsrc2src_1.shape[0], dstsrc2dst_1.shape[0])
    g1, d1 = gather2(x, dstsrc2src_1, dstsrc2dst_1)
    return _mm_small(dif_mat_1, g1, d1, w2a, w2b)


# speedup vs baseline: 1.0333x; 1.0333x over previous
import functools

import jax
import jax.numpy as jnp
from jax import lax
from jax.experimental import pallas as pl
from jax.experimental.pallas import tpu as pltpu
from jax.experimental.pallas import tpu_sc as plsc

NC = 2
NS = 16
NW = NC * NS


def _make_sc_gather(V, D, B_src, B_dst):
    bs, bd = B_src // NW, B_dst // NW
    ch_s = min(bs, 128)
    ncs = bs // ch_s
    ch_d = min(bd, 128)
    ncd = bd // ch_d
    mesh = plsc.VectorSubcoreMesh(core_axis_name="c", subcore_axis_name="s")

    @functools.partial(
        pl.kernel,
        out_type=(
            jax.ShapeDtypeStruct((B_src, D), jnp.float32),
            jax.ShapeDtypeStruct((B_dst, D), jnp.float32),
        ),
        mesh=mesh,
        scratch_types=[
            pltpu.VMEM((ncs, ch_s), jnp.int32),
            pltpu.VMEM((bs, D), jnp.float32),
            pltpu.VMEM((ncd, ch_d), jnp.int32),
            pltpu.VMEM((bd, D), jnp.float32),
            pltpu.SemaphoreType.DMA,
            pltpu.SemaphoreType.DMA,
        ],
    )
    def gather(table, sidx, didx, src_out, dst_out,
               sidx_v, srows_v, didx_v, drows_v, gsem, wsem):
        wid = lax.axis_index("s") * NC + lax.axis_index("c")
        sbase = wid * bs
        dbase = wid * bd
        for j in range(ncs):
            pltpu.sync_copy(sidx.at[pl.ds(sbase + j * ch_s, ch_s)], sidx_v.at[j])
        for j in range(ncd):
            pltpu.sync_copy(didx.at[pl.ds(dbase + j * ch_d, ch_d)], didx_v.at[j])
        gathers = [
            pltpu.async_copy(table.at[sidx_v.at[j]],
                             srows_v.at[pl.ds(j * ch_s, ch_s)], gsem)
            for j in range(ncs)
        ] + [
            pltpu.async_copy(table.at[didx_v.at[j]],
                             drows_v.at[pl.ds(j * ch_d, ch_d)], gsem)
            for j in range(ncd)
        ]
        writebacks = []
        for j in range(ncs):
            gathers[j].wait()
            writebacks.append(pltpu.async_copy(
                srows_v.at[pl.ds(j * ch_s, ch_s)],
                src_out.at[pl.ds(sbase + j * ch_s, ch_s)], wsem))
        for j in range(ncd):
            gathers[ncs + j].wait()
            writebacks.append(pltpu.async_copy(
                drows_v.at[pl.ds(j * ch_d, ch_d)],
                dst_out.at[pl.ds(dbase + j * ch_d, ch_d)], wsem))
        for c in writebacks:
            c.wait()

    return gather


def _mm_big(dif, g, d2, wa, wb, bk=2048):
    M, K = dif.shape
    D = g.shape[1]
    nk = K // bk

    def body(dif_ref, g_ref, d2_ref, wa_ref, wb_ref, out_ref, acc_ref):
        k = pl.program_id(0)

        @pl.when(k == 0)
        def _():
            acc_ref[...] = jnp.zeros_like(acc_ref)

        acc_ref[...] += jnp.dot(dif_ref[...], g_ref[...],
                                preferred_element_type=jnp.float32)

        @pl.when(k == nk - 1)
        def _():
            out_ref[...] = jnp.maximum(
                jnp.dot(acc_ref[...], wa_ref[...],
                        preferred_element_type=jnp.float32)
                + jnp.dot(d2_ref[...], wb_ref[...],
                          preferred_element_type=jnp.float32),
                0.0)

    return pl.pallas_call(
        body,
        grid=(nk,),
        in_specs=[
            pl.BlockSpec((M, bk), lambda k: (0, k)),
            pl.BlockSpec((bk, D), lambda k: (k, 0)),
            pl.BlockSpec((M, D), lambda k: (0, 0)),
            pl.BlockSpec((D, D), lambda k: (0, 0)),
            pl.BlockSpec((D, D), lambda k: (0, 0)),
        ],
        out_specs=pl.BlockSpec((M, D), lambda k: (0, 0)),
        out_shape=jax.ShapeDtypeStruct((M, D), jnp.float32),
        scratch_shapes=[pltpu.VMEM((M, D), jnp.float32)],
    )(dif, g, d2, wa, wb)


def _mm_small(dif, g, d1, wa, wb):
    M = dif.shape[0]
    D = g.shape[1]

    def body(dif_ref, g_ref, d_ref, wa_ref, wb_ref, out_ref):
        agg = jnp.dot(dif_ref[...], g_ref[...],
                      preferred_element_type=jnp.float32)
        out_ref[...] = jnp.maximum(
            jnp.dot(agg, wa_ref[...], preferred_element_type=jnp.float32)
            + jnp.dot(d_ref[...], wb_ref[...],
                      preferred_element_type=jnp.float32),
            0.0)

    return pl.pallas_call(
        body,
        out_shape=jax.ShapeDtypeStruct((M, D), jnp.float32),
    )(dif, g, d1, wa, wb)


def kernel(src_nodes, dstsrc2src_1, dstsrc2src_2, dstsrc2dst_1, dstsrc2dst_2,
           dif_mat_1, dif_mat_2, w1, w2):
    D = src_nodes.shape[1]
    w1a, w1b = w1[:D], w1[D:]
    w2a, w2b = w2[:D], w2[D:]

    gather1 = _make_sc_gather(src_nodes.shape[0], D,
                              dstsrc2src_2.shape[0], dstsrc2dst_2.shape[0])
    g2, d2 = gather1(src_nodes, dstsrc2src_2, dstsrc2dst_2)
    x = _mm_big(dif_mat_2, g2, d2, w1a, w1b)

    gather2 = _make_sc_gather(x.shape[0], D,
                              dstsrc2src_1.shape[0], dstsrc2dst_1.shape[0])
    g1, d1 = gather2(x, dstsrc2src_1, dstsrc2dst_1)
    return _mm_small(dif_mat_1, g1, d1, w2a, w2b)


# speedup vs baseline: 1.0595x; 1.0254x over previous
import functools

import jax
import jax.numpy as jnp
from jax import lax
from jax.experimental import pallas as pl
from jax.experimental.pallas import tpu as pltpu
from jax.experimental.pallas import tpu_sc as plsc

NC = 2
NS = 16
NW = NC * NS


def _make_sc_gather(V, D, B_src, B_dst):
    bs, bd = B_src // NW, B_dst // NW
    ch_s = min(bs, 128)
    ncs = bs // ch_s
    ch_d = min(bd, 128)
    ncd = bd // ch_d
    mesh = plsc.VectorSubcoreMesh(core_axis_name="c", subcore_axis_name="s")

    @functools.partial(
        pl.kernel,
        out_type=(
            jax.ShapeDtypeStruct((B_src, D), jnp.float32),
            jax.ShapeDtypeStruct((B_dst, D), jnp.float32),
        ),
        mesh=mesh,
        scratch_types=[
            pltpu.VMEM((ncs, ch_s), jnp.int32),
            pltpu.VMEM((bs, D), jnp.float32),
            pltpu.VMEM((ncd, ch_d), jnp.int32),
            pltpu.VMEM((bd, D), jnp.float32),
            pltpu.SemaphoreType.DMA,
            pltpu.SemaphoreType.DMA,
        ],
    )
    def gather(table, sidx, didx, src_out, dst_out,
               sidx_v, srows_v, didx_v, drows_v, gsem, wsem):
        wid = lax.axis_index("s") * NC + lax.axis_index("c")
        sbase = wid * bs
        dbase = wid * bd
        for j in range(ncs):
            pltpu.sync_copy(sidx.at[pl.ds(sbase + j * ch_s, ch_s)], sidx_v.at[j])
        for j in range(ncd):
            pltpu.sync_copy(didx.at[pl.ds(dbase + j * ch_d, ch_d)], didx_v.at[j])
        gathers = [
            pltpu.async_copy(table.at[sidx_v.at[j]],
                             srows_v.at[pl.ds(j * ch_s, ch_s)], gsem)
            for j in range(ncs)
        ] + [
            pltpu.async_copy(table.at[didx_v.at[j]],
                             drows_v.at[pl.ds(j * ch_d, ch_d)], gsem)
            for j in range(ncd)
        ]
        writebacks = []
        for j in range(ncs):
            gathers[j].wait()
            writebacks.append(pltpu.async_copy(
                srows_v.at[pl.ds(j * ch_s, ch_s)],
                src_out.at[pl.ds(sbase + j * ch_s, ch_s)], wsem))
        for j in range(ncd):
            gathers[ncs + j].wait()
            writebacks.append(pltpu.async_copy(
                drows_v.at[pl.ds(j * ch_d, ch_d)],
                dst_out.at[pl.ds(dbase + j * ch_d, ch_d)], wsem))
        for c in writebacks:
            c.wait()

    return gather


def _mm_big(dif, g, d2, wa, wb, bk=2048):
    M, K = dif.shape
    D = g.shape[1]
    nk = K // bk

    def body(dif_ref, g_ref, d2_ref, wa_ref, wb_ref, out_ref, acc_ref):
        k = pl.program_id(0)

        @pl.when(k == 0)
        def _():
            acc_ref[...] = jnp.zeros_like(acc_ref)

        acc_ref[...] += jnp.dot(dif_ref[...], g_ref[...],
                                preferred_element_type=jnp.float32)

        @pl.when(k == nk - 1)
        def _():
            out_ref[...] = jnp.maximum(
                jnp.dot(acc_ref[...], wa_ref[...],
                        preferred_element_type=jnp.float32)
                + jnp.dot(d2_ref[...], wb_ref[...],
                          preferred_element_type=jnp.float32),
                0.0)

    return pl.pallas_call(
        body,
        grid=(nk,),
        in_specs=[
            pl.BlockSpec((M, bk), lambda k: (0, k)),
            pl.BlockSpec((bk, D), lambda k: (k, 0)),
            pl.BlockSpec((M, D), lambda k: (0, 0)),
            pl.BlockSpec((D, D), lambda k: (0, 0)),
            pl.BlockSpec((D, D), lambda k: (0, 0)),
        ],
        out_specs=pl.BlockSpec((M, D), lambda k: (0, 0)),
        out_shape=jax.ShapeDtypeStruct((M, D), jnp.float32),
        scratch_shapes=[pltpu.VMEM((M, D), jnp.float32)],
    )(dif, g, d2, wa, wb)


def _mm_small(dif, g, d1, wa, wb):
    M = dif.shape[0]
    D = g.shape[1]

    def body(dif_ref, g_ref, d_ref, wa_ref, wb_ref, out_ref):
        agg = jnp.dot(dif_ref[...], g_ref[...],
                      preferred_element_type=jnp.float32)
        out_ref[...] = jnp.maximum(
            jnp.dot(agg, wa_ref[...], preferred_element_type=jnp.float32)
            + jnp.dot(d_ref[...], wb_ref[...],
                      preferred_element_type=jnp.float32),
            0.0)

    return pl.pallas_call(
        body,
        out_shape=jax.ShapeDtypeStruct((M, D), jnp.float32),
    )(dif, g, d1, wa, wb)


def _bw_probe(dif, bk=2048):
    M, K = dif.shape
    nk = K // bk

    def body(dif_ref, out_ref):
        k = pl.program_id(0)

        @pl.when(k == 0)
        def _():
            out_ref[...] = jnp.zeros_like(out_ref)

        out_ref[...] += dif_ref[:, :128]

    return pl.pallas_call(
        body,
        grid=(nk,),
        in_specs=[pl.BlockSpec((M, bk), lambda k: (0, k))],
        out_specs=pl.BlockSpec((M, 128), lambda k: (0, 0)),
        out_shape=jax.ShapeDtypeStruct((M, 128), jnp.float32),
    )(dif)


def kernel(src_nodes, dstsrc2src_1, dstsrc2src_2, dstsrc2dst_1, dstsrc2dst_2,
           dif_mat_1, dif_mat_2, w1, w2):
    return _kernel_real(src_nodes, dstsrc2src_1, dstsrc2src_2, dstsrc2dst_1,
                        dstsrc2dst_2, dif_mat_1, dif_mat_2, w1, w2)


def _kernel_real(src_nodes, dstsrc2src_1, dstsrc2src_2, dstsrc2dst_1, dstsrc2dst_2,
           dif_mat_1, dif_mat_2, w1, w2):
    D = src_nodes.shape[1]
    w1a, w1b = w1[:D], w1[D:]
    w2a, w2b = w2[:D], w2[D:]

    gather1 = _make_sc_gather(src_nodes.shape[0], D,
                              dstsrc2src_2.shape[0], dstsrc2dst_2.shape[0])
    g2, d2 = gather1(src_nodes, dstsrc2src_2, dstsrc2dst_2)
    x = _mm_big(dif_mat_2, g2, d2, w1a, w1b, bk=1024)

    gather2 = _make_sc_gather(x.shape[0], D,
                              dstsrc2src_1.shape[0], dstsrc2dst_1.shape[0])
    g1, d1 = gather2(x, dstsrc2src_1, dstsrc2dst_1)
    return _mm_small(dif_mat_1, g1, d1, w2a, w2b)
